# baseline (speedup 1.0000x reference)
import functools
import jax
import jax.numpy as jnp
from jax import lax
from jax.experimental import pallas as pl
from jax.experimental.pallas import tpu as pltpu
from jax.experimental.pallas import tpu_sc as plsc

N = 2048
HALF = 1024
DIN = 256
HID = 256
LAT = 64
E = 32768
EY = 16384

NTILES = 32
ROWS_PT = N // NTILES
YCOLS_PT = HALF // NTILES
CLEN = E + 128
CH = 64

_mesh = plsc.VectorSubcoreMesh(core_axis_name="c", subcore_axis_name="s")
_sc_params = pltpu.CompilerParams(needs_layout_passes=False)


def _wid():
    return lax.axis_index("s") * 2 + lax.axis_index("c")


def _sck_a(src_hbm, dst_hbm, ys_hbm, yd_hbm,
           deg_hbm, ayqt_hbm, csrc_hbm, cdloc_hbm, cnt_hbm,
           sbuf, dbuf, hist, slab, csrc_v, cdloc_v, cnt_v, sem):
    wid = _wid()
    dbase = wid * ROWS_PT
    ybase = wid * YCOLS_PT
    iota = lax.iota(jnp.int32, 16)
    zeros16 = jnp.zeros((16,), jnp.float32)
    ones16 = jnp.ones((16,), jnp.float32)

    hist[pl.ds(0, 16)] = zeros16
    hist[pl.ds(16, 16)] = zeros16
    hist[pl.ds(32, 16)] = zeros16
    hist[pl.ds(48, 16)] = zeros16

    @plsc.parallel_loop(0, YCOLS_PT * 64, unroll=4)
    def _(i):
        slab[i // 64, pl.ds((i % 64) * 16, 16)] = zeros16

    CHUNK = 2048

    def echunk(ci, ptr):
        pltpu.sync_copy(src_hbm.at[pl.ds(ci * CHUNK, CHUNK)], sbuf)
        pltpu.sync_copy(dst_hbm.at[pl.ds(ci * CHUNK, CHUNK)], dbuf)

        def grp(g, p):
            d = dbuf[pl.ds(g * 16, 16)]
            s = sbuf[pl.ds(g * 16, 16)]
            dloc = d - dbase
            m = (d >= dbase) & (d < dbase + ROWS_PT)
            plsc.addupdate_scatter(hist, [dloc], ones16, mask=m)
            plsc.store_compressed(csrc_v.at[pl.ds(p, 16)], s, mask=m)
            plsc.store_compressed(cdloc_v.at[pl.ds(p, 16)], dloc, mask=m)
            npop = plsc.all_reduce_population_count(m)
            return p + lax.reduce_max(npop, (0,))
        return lax.fori_loop(0, CHUNK // 16, grp, ptr)

    ptr = lax.fori_loop(0, E // CHUNK, echunk, jnp.int32(0))

    n2 = jnp.maximum(((ptr + 127) // 128) * 128, 128)
    sent_d = jnp.full((16,), ROWS_PT, jnp.int32)
    sent_s = jnp.zeros((16,), jnp.int32)

    def pad(i, _):
        @pl.when(ptr + i * 16 < n2)
        def _():
            csrc_v[pl.ds(ptr + i * 16, 16)] = sent_s
            cdloc_v[pl.ds(ptr + i * 16, 16)] = sent_d
        return 0
    lax.fori_loop(0, 8, pad, 0)

    def ychunk(ci, _):
        pltpu.sync_copy(ys_hbm.at[pl.ds(ci * CHUNK, CHUNK)], sbuf)
        pltpu.sync_copy(yd_hbm.at[pl.ds(ci * CHUNK, CHUNK)], dbuf)

        @plsc.parallel_loop(0, CHUNK // 16, unroll=2)
        def grp(g):
            r = sbuf[pl.ds(g * 16, 16)]
            col = dbuf[pl.ds(g * 16, 16)]
            cloc = col - ybase
            m = (r < HALF) & (col >= ybase) & (col < ybase + YCOLS_PT)
            plsc.addupdate_scatter(slab, [cloc, r], ones16, mask=m)
        return 0
    lax.fori_loop(0, EY // CHUNK, ychunk, 0)

    pltpu.sync_copy(hist, deg_hbm.at[pl.ds(dbase, ROWS_PT)])
    pltpu.sync_copy(slab, ayqt_hbm.at[pl.ds(ybase, YCOLS_PT)])
    pltpu.sync_copy(csrc_v, csrc_hbm.at[wid])
    pltpu.sync_copy(cdloc_v, cdloc_hbm.at[wid])
    cnt_v[...] = jnp.broadcast_to(n2, (16,)).astype(jnp.int32)
    pltpu.sync_copy(cnt_v, cnt_hbm.at[wid])


def _run_sck_a(src, dst, ys, yd):
    f = pl.kernel(
        _sck_a,
        out_type=[
            jax.ShapeDtypeStruct((N,), jnp.float32),
            jax.ShapeDtypeStruct((HALF, HALF), jnp.float32),
            jax.ShapeDtypeStruct((NTILES, CLEN), jnp.int32),
            jax.ShapeDtypeStruct((NTILES, CLEN), jnp.int32),
            jax.ShapeDtypeStruct((NTILES, 16), jnp.int32),
        ],
        mesh=_mesh,
        compiler_params=_sc_params,
        scratch_types=[
            pltpu.VMEM((2048,), jnp.int32),
            pltpu.VMEM((2048,), jnp.int32),
            pltpu.VMEM((ROWS_PT,), jnp.float32),
            pltpu.VMEM((YCOLS_PT, HALF), jnp.float32),
            pltpu.VMEM((CLEN,), jnp.int32),
            pltpu.VMEM((CLEN,), jnp.int32),
            pltpu.VMEM((16,), jnp.int32),
            pltpu.SemaphoreType.DMA,
        ],
    )
    return f(src, dst, ys, yd)


def _make_sck_b(D):
    def body(v_hbm, csrc_hbm, cdloc_hbm, cnt_hbm, out_hbm,
             csrc_v, cdloc_v, cnt_v, acc, rb0, rb1b, sem0, sem1):
        wid = _wid()
        dbase = wid * ROWS_PT
        iota = lax.iota(jnp.int32, 16)
        zeros16 = jnp.zeros((16,), jnp.float32)

        @plsc.parallel_loop(0, (ROWS_PT + 1) * (D // 16), unroll=4)
        def _(i):
            acc[pl.ds(i * 16, 16)] = zeros16

        pltpu.sync_copy(csrc_hbm.at[wid], csrc_v)
        pltpu.sync_copy(cdloc_hbm.at[wid], cdloc_v)
        pltpu.sync_copy(cnt_hbm.at[wid], cnt_v)
        n2 = lax.reduce_max(cnt_v[...], (0,))

        def fire(buf, sem, base):
            b = pl.multiple_of(base, CH)
            pltpu.async_copy(v_hbm.at[csrc_v.at[pl.ds(b, CH)]], buf, sem)

        def wait(buf, sem):
            pltpu.make_async_copy(v_hbm.at[csrc_v.at[pl.ds(0, CH)]], buf, sem).wait()

        def acc_chunk(buf, base):
            @plsc.parallel_loop(0, CH // 16)
            def _(k):
                dl_vec = cdloc_v[pl.ds(base + k * 16, 16)]
                for j in range(16):
                    dj = lax.reduce_sum(
                        jnp.where(iota == j, dl_vec, jnp.zeros_like(dl_vec)),
                        (0,))
                    rb = pl.multiple_of(dj * D, 8)
                    for q in range(D // 16):
                        plsc.addupdate(acc.at[pl.ds(rb + q * 16, 16)],
                                       buf[k * 16 + j, pl.ds(q * 16, 16)])

        fire(rb0, sem0, 0)

        def pair(i, _):
            @pl.when(i + CH < n2)
            def _():
                fire(rb1b, sem1, i + CH)
            wait(rb0, sem0)
            acc_chunk(rb0, i)
            @pl.when(i + 2 * CH < n2)
            def _():
                fire(rb0, sem0, i + 2 * CH)
            @pl.when(i + CH < n2)
            def _():
                wait(rb1b, sem1)
                acc_chunk(rb1b, i + CH)
            return 0
        lax.while_loop(lambda st: st < n2,
                       lambda st: (pair(st, 0), st + 2 * CH)[1],
                       jnp.int32(0))

        pltpu.sync_copy(acc.at[pl.ds(0, ROWS_PT * D)],
                        out_hbm.at[pl.ds(dbase * D, ROWS_PT * D)])

    def run(v, csrc, cdloc, cnt):
        f = pl.kernel(
            body,
            out_type=jax.ShapeDtypeStruct((N * D,), jnp.float32),
            mesh=_mesh,
            compiler_params=_sc_params,
            scratch_types=[
                pltpu.VMEM((CLEN,), jnp.int32),
                pltpu.VMEM((CLEN,), jnp.int32),
                pltpu.VMEM((16,), jnp.int32),
                pltpu.VMEM(((ROWS_PT + 1) * D,), jnp.float32),
                pltpu.VMEM((CH, D), jnp.float32),
                pltpu.VMEM((CH, D), jnp.float32),
                pltpu.SemaphoreType.DMA,
                pltpu.SemaphoreType.DMA,
            ],
        )
        return f(v, csrc, cdloc, cnt).reshape(N, D)
    return run


_sck_b_256 = _make_sck_b(HID)
_sck_b_128 = _make_sck_b(128)


def _tck_a(deg_ref, cnt_ref, x_ref, w1_ref, ayq_ref, dinv_ref, v1_ref):
    cnt = cnt_ref[...]
    ayq = jnp.where(cnt != 0.0, jax.nn.sigmoid(cnt), 0.0)
    ayq_ref[...] = ayq
    ydeg = jnp.sum(ayq, axis=1, keepdims=True)
    deg = deg_ref[...] + 1.0 + jnp.concatenate(
        [ydeg, jnp.zeros((N - HALF, 1), jnp.float32)], axis=0)
    dinv = lax.rsqrt(deg)
    dinv_ref[...] = dinv
    xw = jnp.dot(x_ref[...], w1_ref[...], preferred_element_type=jnp.float32)
    v1_ref[...] = xw * dinv


def _tck_b(e1_ref, v1_ref, ayq_ref, dinv_ref, b1_ref, wmu_ref, v2_ref):
    v1 = v1_ref[...]
    yq = jnp.dot(ayq_ref[...], v1[:HALF], preferred_element_type=jnp.float32)
    s = e1_ref[...] + v1 + jnp.concatenate(
        [yq, jnp.zeros((N - HALF, HID), jnp.float32)], axis=0)
    dinv_col = dinv_ref[...]
    h = jnp.maximum(s * dinv_col + b1_ref[...], 0.0)
    v2 = jnp.dot(h, wmu_ref[...], preferred_element_type=jnp.float32) * dinv_col
    v2_ref[...] = jnp.concatenate(
        [v2, jnp.zeros((N, 128 - LAT), jnp.float32)], axis=1)


def _tck_c(e2_ref, v2_ref, ayq_ref, dinv_ref, bmu_ref, mu_ref):
    v2 = v2_ref[...]
    yq = jnp.dot(ayq_ref[...], v2[:HALF], preferred_element_type=jnp.float32)
    s = e2_ref[...] + v2 + jnp.concatenate(
        [yq, jnp.zeros((N - HALF, 128), jnp.float32)], axis=0)
    mu_ref[...] = (s * dinv_ref[...])[:, :LAT] + bmu_ref[...]


def kernel(x, edge_index, y_edge_index, Wb, pW1, pb1, pWmu, pbmu, pWls, pbls,
           rW1, rb1, rWmu, rbmu, rWls, rbls):
    src = edge_index[0]
    dst = edge_index[1]
    ys = y_edge_index[0]
    yd = y_edge_index[1]

    deg_base, ayqt_cnt, csrc, cdloc, cnt = _run_sck_a(src, dst, ys, yd)

    ayq, dinv, v1 = pl.pallas_call(
        _tck_a,
        out_shape=[
            jax.ShapeDtypeStruct((HALF, HALF), jnp.float32),
            jax.ShapeDtypeStruct((N, 1), jnp.float32),
            jax.ShapeDtypeStruct((N, HID), jnp.float32),
        ],
    )(deg_base.reshape(N, 1), ayqt_cnt, x, rW1)

    e1 = _sck_b_256(v1, csrc, cdloc, cnt)
    v2 = pl.pallas_call(
        _tck_b, out_shape=jax.ShapeDtypeStruct((N, 128), jnp.float32),
    )(e1, v1, ayq, dinv, rb1.reshape(1, HID), rWmu)

    e2 = _sck_b_128(v2, csrc, cdloc, cnt)
    mu = pl.pallas_call(
        _tck_c, out_shape=jax.ShapeDtypeStruct((N, LAT), jnp.float32),
    )(e2, v2, ayq, dinv, rbmu.reshape(1, LAT))
    return mu


# speedup vs baseline: 1.0088x; 1.0088x over previous
import functools
import jax
import jax.numpy as jnp
from jax import lax
from jax.experimental import pallas as pl
from jax.experimental.pallas import tpu as pltpu
from jax.experimental.pallas import tpu_sc as plsc

N = 2048
HALF = 1024
DIN = 256
HID = 256
LAT = 64
E = 32768
EY = 16384

NTILES = 32
ROWS_PT = N // NTILES
YCOLS_PT = HALF // NTILES
CLEN = E + 128
CH = 64

_mesh = plsc.VectorSubcoreMesh(core_axis_name="c", subcore_axis_name="s")
_sc_params = pltpu.CompilerParams(needs_layout_passes=False)


def _wid():
    return lax.axis_index("s") * 2 + lax.axis_index("c")


def _sck_a(src_hbm, dst_hbm, ys_hbm, yd_hbm,
           deg_hbm, ayqt_hbm, csrc_hbm, cdloc_hbm, cnt_hbm,
           sbuf, dbuf, hist, slab, csrc_v, cdloc_v, cnt_v, sem):
    wid = _wid()
    dbase = wid * ROWS_PT
    ybase = wid * YCOLS_PT
    iota = lax.iota(jnp.int32, 16)
    zeros16 = jnp.zeros((16,), jnp.float32)
    ones16 = jnp.ones((16,), jnp.float32)

    hist[pl.ds(0, 16)] = zeros16
    hist[pl.ds(16, 16)] = zeros16
    hist[pl.ds(32, 16)] = zeros16
    hist[pl.ds(48, 16)] = zeros16

    @plsc.parallel_loop(0, YCOLS_PT * 64, unroll=4)
    def _(i):
        slab[i // 64, pl.ds((i % 64) * 16, 16)] = zeros16

    CHUNK = 2048

    def echunk(ci, ptr):
        pltpu.sync_copy(src_hbm.at[pl.ds(ci * CHUNK, CHUNK)], sbuf)
        pltpu.sync_copy(dst_hbm.at[pl.ds(ci * CHUNK, CHUNK)], dbuf)

        def grp(g, p):
            d = dbuf[pl.ds(g * 16, 16)]
            s = sbuf[pl.ds(g * 16, 16)]
            dloc = d - dbase
            m = (d >= dbase) & (d < dbase + ROWS_PT)
            plsc.addupdate_scatter(hist, [dloc], ones16, mask=m)
            plsc.store_compressed(csrc_v.at[pl.ds(p, 16)], s, mask=m)
            plsc.store_compressed(cdloc_v.at[pl.ds(p, 16)], dloc, mask=m)
            npop = plsc.all_reduce_population_count(m)
            return p + lax.reduce_max(npop, (0,))
        return lax.fori_loop(0, CHUNK // 16, grp, ptr)

    ptr = lax.fori_loop(0, E // CHUNK, echunk, jnp.int32(0))

    n2 = jnp.maximum(((ptr + 127) // 128) * 128, 128)
    sent_d = jnp.full((16,), ROWS_PT, jnp.int32)
    sent_s = jnp.zeros((16,), jnp.int32)

    def pad(i, _):
        @pl.when(ptr + i * 16 < n2)
        def _():
            csrc_v[pl.ds(ptr + i * 16, 16)] = sent_s
            cdloc_v[pl.ds(ptr + i * 16, 16)] = sent_d
        return 0
    lax.fori_loop(0, 8, pad, 0)

    def ychunk(ci, _):
        pltpu.sync_copy(ys_hbm.at[pl.ds(ci * CHUNK, CHUNK)], sbuf)
        pltpu.sync_copy(yd_hbm.at[pl.ds(ci * CHUNK, CHUNK)], dbuf)

        @plsc.parallel_loop(0, CHUNK // 16, unroll=2)
        def grp(g):
            r = sbuf[pl.ds(g * 16, 16)]
            col = dbuf[pl.ds(g * 16, 16)]
            cloc = col - ybase
            m = (r < HALF) & (col >= ybase) & (col < ybase + YCOLS_PT)
            plsc.addupdate_scatter(slab, [cloc, r], ones16, mask=m)
        return 0
    lax.fori_loop(0, EY // CHUNK, ychunk, 0)

    pltpu.sync_copy(hist, deg_hbm.at[pl.ds(dbase, ROWS_PT)])
    pltpu.sync_copy(slab, ayqt_hbm.at[pl.ds(ybase, YCOLS_PT)])
    pltpu.sync_copy(csrc_v, csrc_hbm.at[wid])
    pltpu.sync_copy(cdloc_v, cdloc_hbm.at[wid])
    cnt_v[...] = jnp.broadcast_to(n2, (16,)).astype(jnp.int32)
    pltpu.sync_copy(cnt_v, cnt_hbm.at[wid])


def _run_sck_a(src, dst, ys, yd):
    f = pl.kernel(
        _sck_a,
        out_type=[
            jax.ShapeDtypeStruct((N,), jnp.float32),
            jax.ShapeDtypeStruct((HALF, HALF), jnp.float32),
            jax.ShapeDtypeStruct((NTILES, CLEN), jnp.int32),
            jax.ShapeDtypeStruct((NTILES, CLEN), jnp.int32),
            jax.ShapeDtypeStruct((NTILES, 16), jnp.int32),
        ],
        mesh=_mesh,
        compiler_params=_sc_params,
        scratch_types=[
            pltpu.VMEM((2048,), jnp.int32),
            pltpu.VMEM((2048,), jnp.int32),
            pltpu.VMEM((ROWS_PT,), jnp.float32),
            pltpu.VMEM((YCOLS_PT, HALF), jnp.float32),
            pltpu.VMEM((CLEN,), jnp.int32),
            pltpu.VMEM((CLEN,), jnp.int32),
            pltpu.VMEM((16,), jnp.int32),
            pltpu.SemaphoreType.DMA,
        ],
    )
    return f(src, dst, ys, yd)


def _make_sck_b(D):
    def body(v_hbm, csrc_hbm, cdloc_hbm, cnt_hbm, out_hbm,
             csrc_v, cdloc_v, cnt_v, acc, rb0, rb1b, sem0, sem1):
        wid = _wid()
        dbase = wid * ROWS_PT
        iota = lax.iota(jnp.int32, 16)
        zeros16 = jnp.zeros((16,), jnp.float32)

        @plsc.parallel_loop(0, (ROWS_PT + 1) * (D // 16), unroll=4)
        def _(i):
            acc[pl.ds(i * 16, 16)] = zeros16

        pltpu.sync_copy(csrc_hbm.at[wid], csrc_v)
        pltpu.sync_copy(cdloc_hbm.at[wid], cdloc_v)
        pltpu.sync_copy(cnt_hbm.at[wid], cnt_v)
        n2 = lax.reduce_max(cnt_v[...], (0,))

        def fire(buf, sem, base):
            b = pl.multiple_of(base, CH)
            pltpu.async_copy(v_hbm.at[csrc_v.at[pl.ds(b, CH)]], buf, sem)

        def wait(buf, sem):
            pltpu.make_async_copy(v_hbm.at[csrc_v.at[pl.ds(0, CH)]], buf, sem).wait()

        def acc_chunk(buf, base):
            @plsc.parallel_loop(0, CH // 16)
            def _(k):
                dl_vec = cdloc_v[pl.ds(base + k * 16, 16)]
                for j in range(16):
                    dj = lax.reduce_sum(
                        jnp.where(iota == j, dl_vec, jnp.zeros_like(dl_vec)),
                        (0,))
                    rb = pl.multiple_of(dj * D, 8)
                    for q in range(D // 16):
                        plsc.addupdate(acc.at[pl.ds(rb + q * 16, 16)],
                                       buf[k * 16 + j, pl.ds(q * 16, 16)])

        fire(rb0, sem0, 0)

        def pair(i, _):
            @pl.when(i + CH < n2)
            def _():
                fire(rb1b, sem1, i + CH)
            wait(rb0, sem0)
            acc_chunk(rb0, i)
            @pl.when(i + 2 * CH < n2)
            def _():
                fire(rb0, sem0, i + 2 * CH)
            @pl.when(i + CH < n2)
            def _():
                wait(rb1b, sem1)
                acc_chunk(rb1b, i + CH)
            return 0
        lax.while_loop(lambda st: st < n2,
                       lambda st: (pair(st, 0), st + 2 * CH)[1],
                       jnp.int32(0))

        pltpu.sync_copy(acc.at[pl.ds(0, ROWS_PT * D)],
                        out_hbm.at[pl.ds(dbase * D, ROWS_PT * D)])

    def run(v, csrc, cdloc, cnt):
        f = pl.kernel(
            body,
            out_type=jax.ShapeDtypeStruct((N * D,), jnp.float32),
            mesh=_mesh,
            compiler_params=_sc_params,
            scratch_types=[
                pltpu.VMEM((CLEN,), jnp.int32),
                pltpu.VMEM((CLEN,), jnp.int32),
                pltpu.VMEM((16,), jnp.int32),
                pltpu.VMEM(((ROWS_PT + 1) * D,), jnp.float32),
                pltpu.VMEM((CH, D), jnp.float32),
                pltpu.VMEM((CH, D), jnp.float32),
                pltpu.SemaphoreType.DMA,
                pltpu.SemaphoreType.DMA,
            ],
        )
        return f(v, csrc, cdloc, cnt).reshape(N, D)
    return run


_sck_b_256 = _make_sck_b(HID)
_sck_b_128 = _make_sck_b(128)


def _tck_a(deg_ref, cnt_ref, x_ref, w1_ref, ayq_ref, dinv_ref, v1_ref):
    cnt = cnt_ref[...]
    ayq = jnp.where(cnt != 0.0, jax.nn.sigmoid(cnt), 0.0)
    ayq_ref[...] = ayq
    ydeg = jnp.sum(ayq, axis=1, keepdims=True)
    deg = deg_ref[...] + 1.0 + jnp.concatenate(
        [ydeg, jnp.zeros((N - HALF, 1), jnp.float32)], axis=0)
    dinv = lax.rsqrt(deg)
    dinv_ref[...] = dinv
    xw = jnp.dot(x_ref[...], w1_ref[...], preferred_element_type=jnp.float32)
    v1_ref[...] = xw * dinv


def _tck_y(ayq_ref, v_ref, yq_ref):
    yq_ref[...] = jnp.dot(ayq_ref[...], v_ref[...][:HALF],
                          preferred_element_type=jnp.float32)


def _tck_b(e1_ref, v1_ref, yq_ref, dinv_ref, b1_ref, wmu_ref, v2_ref):
    v1 = v1_ref[...]
    s = e1_ref[...] + v1 + jnp.concatenate(
        [yq_ref[...], jnp.zeros((N - HALF, HID), jnp.float32)], axis=0)
    dinv_col = dinv_ref[...]
    h = jnp.maximum(s * dinv_col + b1_ref[...], 0.0)
    v2 = jnp.dot(h, wmu_ref[...], preferred_element_type=jnp.float32) * dinv_col
    v2_ref[...] = jnp.concatenate(
        [v2, jnp.zeros((N, 128 - LAT), jnp.float32)], axis=1)


def _tck_c(e2_ref, v2_ref, yq_ref, dinv_ref, bmu_ref, mu_ref):
    v2 = v2_ref[...]
    s = e2_ref[...] + v2 + jnp.concatenate(
        [yq_ref[...], jnp.zeros((N - HALF, 128), jnp.float32)], axis=0)
    mu_ref[...] = (s * dinv_ref[...])[:, :LAT] + bmu_ref[...]


def kernel(x, edge_index, y_edge_index, Wb, pW1, pb1, pWmu, pbmu, pWls, pbls,
           rW1, rb1, rWmu, rbmu, rWls, rbls):
    src = edge_index[0]
    dst = edge_index[1]
    ys = y_edge_index[0]
    yd = y_edge_index[1]

    deg_base, ayqt_cnt, csrc, cdloc, cnt = _run_sck_a(src, dst, ys, yd)

    ayq, dinv, v1 = pl.pallas_call(
        _tck_a,
        out_shape=[
            jax.ShapeDtypeStruct((HALF, HALF), jnp.float32),
            jax.ShapeDtypeStruct((N, 1), jnp.float32),
            jax.ShapeDtypeStruct((N, HID), jnp.float32),
        ],
    )(deg_base.reshape(N, 1), ayqt_cnt, x, rW1)

    e1 = _sck_b_256(v1, csrc, cdloc, cnt)
    yq1 = pl.pallas_call(
        _tck_y, out_shape=jax.ShapeDtypeStruct((HALF, HID), jnp.float32),
    )(ayq, v1)

    v2 = pl.pallas_call(
        _tck_b, out_shape=jax.ShapeDtypeStruct((N, 128), jnp.float32),
    )(e1, v1, yq1, dinv, rb1.reshape(1, HID), rWmu)

    e2 = _sck_b_128(v2, csrc, cdloc, cnt)
    yq2 = pl.pallas_call(
        _tck_y, out_shape=jax.ShapeDtypeStruct((HALF, 128), jnp.float32),
    )(ayq, v2)

    mu = pl.pallas_call(
        _tck_c, out_shape=jax.ShapeDtypeStruct((N, LAT), jnp.float32),
    )(e2, v2, yq2, dinv, rbmu.reshape(1, LAT))
    return mu
